# baseline (speedup 1.0000x reference)
import functools

import jax
import jax.numpy as jnp
from jax import lax
from jax.experimental import pallas as pl
from jax.experimental.pallas import tpu as pltpu
from jax.experimental.pallas import tpu_sc as plsc

N = 2097152
NC = 2
NS = 16
NW = NC * NS
CHUNK = N // NW
LANES = 16
_U = 8
HALF = CHUNK // 2
SUB = CHUNK // 4
SUB_GROUPS = SUB // (_U * LANES)

_mesh = plsc.VectorSubcoreMesh(core_axis_name="c", subcore_axis_name="s")
_params = pltpu.CompilerParams(needs_layout_passes=False)


def _wid():
    return lax.axis_index("c") * NS + lax.axis_index("s")


_TC_BL = 8192


def _tc_sums_body(x_ref, o_ref):
    @pl.when(pl.program_id(0) == 0)
    def _():
        o_ref[...] = jnp.zeros_like(o_ref)

    o_ref[...] += jnp.sum(x_ref[...], axis=1)


_chunk_sums_tc = pl.pallas_call(
    _tc_sums_body,
    grid=(CHUNK // _TC_BL,),
    in_specs=[pl.BlockSpec((NW, _TC_BL), lambda i: (0, i))],
    out_specs=pl.BlockSpec((NW,), lambda i: (0,)),
    out_shape=jax.ShapeDtypeStruct((NW,), jnp.float32),
)


@functools.partial(
    pl.kernel,
    out_type=jax.ShapeDtypeStruct((N,), jnp.float32),
    mesh=_mesh,
    compiler_params=_params,
    scratch_types=[
        pltpu.VMEM((SUB,), jnp.float32),
        pltpu.VMEM((SUB,), jnp.float32),
        pltpu.VMEM((SUB,), jnp.float32),
        pltpu.VMEM((SUB,), jnp.float32),
        pltpu.VMEM((NW,), jnp.float32),
        pltpu.SemaphoreType.DMA,
        pltpu.SemaphoreType.DMA,
        pltpu.SemaphoreType.DMA,
        pltpu.SemaphoreType.DMA,
    ],
)
def _scan_chunks(x_hbm, sums_hbm, out_hbm, in0, in1, out0, out1, sums_v,
                 isem0, isem1, osem0, osem1):
    wid = _wid()
    base = wid * CHUNK
    ins = (in0, in1)
    outs = (out0, out1)
    isems = (isem0, isem1)
    osems = (osem0, osem1)

    pltpu.sync_copy(sums_hbm, sums_v)

    in_copies = [None] * 4
    out_copies = [None] * 4
    for b in range(2):
        in_copies[b] = pltpu.async_copy(
            x_hbm.at[pl.ds(base + b * SUB, SUB)], ins[b], isems[b])

    lane = lax.iota(jnp.int32, LANES)
    zv = jnp.zeros((LANES,), jnp.float32)
    v0 = jnp.where(lane < wid, sums_v[pl.ds(0, LANES)], zv)
    v1 = jnp.where(lane + LANES < wid, sums_v[pl.ds(LANES, LANES)], zv)
    carry = jnp.sum(v0 + v1) - 1.0

    for b in range(4):
        in_copies[b].wait()
        if b >= 2:
            out_copies[b - 2].wait()
        ibuf = ins[b % 2]
        obuf = outs[b % 2]

        @plsc.parallel_loop(0, SUB_GROUPS, carry=carry)
        def body(g, c):
            o = g * (_U * LANES)
            ss = []
            ts = []
            for j in range(_U):
                v = ibuf[pl.ds(o + j * LANES, LANES)]
                s = jnp.cumsum(v)
                ss.append(s)
                ts.append(s[15])
            t01 = ts[0] + ts[1]
            t23 = ts[2] + ts[3]
            t45 = ts[4] + ts[5]
            t67 = ts[6] + ts[7]
            t03 = t01 + t23
            e = [None] * _U
            e[1] = ts[0]
            e[2] = t01
            e[3] = t01 + ts[2]
            e[4] = t03
            e[5] = t03 + ts[4]
            e[6] = t03 + t45
            e[7] = e[6] + ts[6]
            obuf[pl.ds(o, LANES)] = ss[0] + c
            for j in range(1, _U):
                obuf[pl.ds(o + j * LANES, LANES)] = ss[j] + (c + e[j])
            return c + (t03 + (t45 + t67))

        carry = body
        out_copies[b] = pltpu.async_copy(
            obuf, out_hbm.at[pl.ds(base + b * SUB, SUB)], osems[b % 2])
        if b + 2 < 4:
            in_copies[b + 2] = pltpu.async_copy(
                x_hbm.at[pl.ds(base + (b + 2) * SUB, SUB)], ins[b % 2], isems[b % 2])

    out_copies[2].wait()
    out_copies[3].wait()


def kernel(mask_i):
    sums = _chunk_sums_tc(mask_i.reshape(NW, CHUNK))
    return _scan_chunks(mask_i, sums)
